# baseline (speedup 1.0000x reference)
import functools

import jax
import jax.numpy as jnp
from jax import lax
from jax.experimental import pallas as pl
from jax.experimental.pallas import tpu as pltpu
from jax.experimental.pallas import tpu_sc as plsc

T = 2048
D = 2048
E = 8
F = 128

TT = 128
G = T // TT + E
T_PAD = G * TT

NC = 2
NS = 16
NW = NC * NS
L = 16
SLOTS_W = T_PAD // NW
CH = SLOTS_W // L
DTOK_W = T // NW


def _router_body(x_ref, cn_ref, dest_ref, eot_ref):
    cn = cn_ref[...]
    cn = cn / (jnp.sqrt(jnp.sum(cn * cn, axis=-1, keepdims=True)) + 1e-8)
    cnt = cn.T

    nchunk = T // TT
    ri = lax.broadcasted_iota(jnp.int32, (TT, TT), 0)
    ci = lax.broadcasted_iota(jnp.int32, (TT, TT), 1)
    Lt = jnp.where(ri >= ci, 1.0, 0.0).astype(jnp.float32)

    ohs = []
    incs = []
    tots = []
    for c in range(nchunk):
        xc = x_ref[pl.ds(c * TT, TT), :]
        xn = xc / (jnp.sqrt(jnp.sum(xc * xc, axis=-1, keepdims=True)) + 1e-8)
        sims = jnp.dot(xn, cnt, preferred_element_type=jnp.float32)
        mx = jnp.max(sims, axis=-1, keepdims=True)
        eidx = lax.broadcasted_iota(jnp.int32, (TT, E), 1)
        idx = jnp.min(jnp.where(sims >= mx, eidx, E), axis=-1, keepdims=True)
        oh = jnp.where(eidx == idx, 1.0, 0.0).astype(jnp.float32)
        inc = jnp.dot(Lt, oh, preferred_element_type=jnp.float32)
        ohs.append(oh)
        incs.append(inc)
        tots.append(inc[TT - 1:TT, :])

    tot = jnp.concatenate(tots, axis=0)
    ri2 = lax.broadcasted_iota(jnp.int32, (nchunk, nchunk), 0)
    ci2 = lax.broadcasted_iota(jnp.int32, (nchunk, nchunk), 1)
    Ls = jnp.where(ri2 > ci2, 1.0, 0.0).astype(jnp.float32)
    ex = jnp.dot(Ls, tot, preferred_element_type=jnp.float32)
    counts = ex[nchunk - 1:nchunk, :] + tot[nchunk - 1:nchunk, :]

    counts_i = (counts + 0.5).astype(jnp.int32)
    tiles_i = (counts_i + (TT - 1)) >> 7
    tiles_f = tiles_i.astype(jnp.float32)
    ri3 = lax.broadcasted_iota(jnp.int32, (E, E), 0)
    ci3 = lax.broadcasted_iota(jnp.int32, (E, E), 1)
    Mu = jnp.where(ri3 <= ci3, 1.0, 0.0).astype(jnp.float32)
    cum_tiles = jnp.dot(tiles_f, Mu, preferred_element_type=jnp.float32)
    poff = (cum_tiles - tiles_f) * float(TT)

    for c in range(nchunk):
        base = poff + ex[c:c + 1, :]
        dvals = jnp.sum(ohs[c] * (base + incs[c] - 1.0), axis=-1, keepdims=True)
        di = dvals.astype(jnp.int32)
        di = (di & ~127) | ((di & 127) ^ 83)
        dest_ref[pl.ds(c * TT, TT), :] = di

    gio = lax.broadcasted_iota(jnp.int32, (G, E), 0).astype(jnp.float32)
    ge = jnp.where(gio >= cum_tiles, 1.0, 0.0)
    eot = jnp.sum(ge, axis=-1, keepdims=True).astype(jnp.int32)
    eot_ref[...] = jnp.minimum(eot, E - 1)


def _route(x, centroids):
    return pl.pallas_call(
        _router_body,
        grid=(1,),
        in_specs=[
            pl.BlockSpec((T, D), lambda i: (0, 0)),
            pl.BlockSpec((E, D), lambda i: (0, 0)),
        ],
        out_specs=[
            pl.BlockSpec((T, 1), lambda i: (0, 0)),
            pl.BlockSpec((G, 1), lambda i: (0, 0)),
        ],
        out_shape=[
            jax.ShapeDtypeStruct((T, 1), jnp.int32),
            jax.ShapeDtypeStruct((G, 1), jnp.int32),
        ],
    )(x, centroids)


NB = 3
TW = T // NW
CT = TW // L


def _sc_gather_body(x_hbm, dest_hbm, xg_hbm,
                    dest_v, dest2_v, rows_v,
                    gsem0, gsem1, gsem2, wsem0, wsem1, wsem2):
    wid = lax.axis_index("c") * NS + lax.axis_index("s")
    lo = wid * TW
    gsems = [gsem0, gsem1, gsem2]
    wsems = [wsem0, wsem1, wsem2]

    pltpu.sync_copy(dest_hbm.at[pl.ds(lo, TW)], dest_v)
    for c in range(CT):
        dest2_v[c, :] = dest_v[pl.ds(c * L, L)]

    gcopies = [None] * CT
    wcopies = [None] * CT
    for c in range(NB):
        gcopies[c] = pltpu.async_copy(x_hbm.at[pl.ds(lo + c * L, L)],
                                      rows_v.at[c % NB], gsems[c % NB])
    for c in range(CT):
        b = c % NB
        gcopies[c].wait()
        wcopies[c] = pltpu.async_copy(rows_v.at[b],
                                      xg_hbm.at[dest2_v.at[c]], wsems[b])
        if c + NB < CT:
            wcopies[c].wait()
            gcopies[c + NB] = pltpu.async_copy(
                x_hbm.at[pl.ds(lo + (c + NB) * L, L)], rows_v.at[b], gsems[b])
    for c in range(max(CT - NB, 0), CT):
        if wcopies[c] is not None:
            wcopies[c].wait()


def _sc_gather(x, dest):
    mesh = plsc.VectorSubcoreMesh(core_axis_name="c", subcore_axis_name="s")
    f = functools.partial(
        pl.kernel,
        mesh=mesh,
        compiler_params=pltpu.CompilerParams(needs_layout_passes=False),
        out_type=jax.ShapeDtypeStruct((T_PAD, D), jnp.float32),
        scratch_types=[
            pltpu.VMEM((TW,), jnp.int32),
            pltpu.VMEM((CT, L), jnp.int32),
            pltpu.VMEM((NB, L, D), jnp.float32),
            pltpu.SemaphoreType.DMA,
            pltpu.SemaphoreType.DMA,
            pltpu.SemaphoreType.DMA,
            pltpu.SemaphoreType.DMA,
            pltpu.SemaphoreType.DMA,
            pltpu.SemaphoreType.DMA,
        ],
    )(_sc_gather_body)
    return f(x, dest)


def _expert_body(eot_ref, xg_ref, W1_ref, b1_ref, W2_ref, b2_ref,
                 lng_ref, lnb_ref, alpha_ref, out_ref):
    x = xg_ref[...]
    xb = x.astype(jnp.bfloat16)
    h = jnp.dot(xb, W1_ref[0], preferred_element_type=jnp.float32) + b1_ref[0]
    h = 0.5 * h * (1.0 + lax.erf(h * 0.7071067811865476))
    hb = h.astype(jnp.bfloat16)
    y = jnp.dot(hb, W2_ref[0], preferred_element_type=jnp.float32) + b2_ref[0]
    mu = jnp.mean(y, axis=-1, keepdims=True)
    yc = y - mu
    var = jnp.mean(yc * yc, axis=-1, keepdims=True)
    y_ln = yc * lax.rsqrt(var + 1e-5) * lng_ref[0] + lnb_ref[0]
    out_ref[...] = x + alpha_ref[0] * y_ln


def _expert_mlp(eot, xg, W1, b1, W2, b2, ln_g, ln_b, alpha_v):
    grid_spec = pltpu.PrefetchScalarGridSpec(
        num_scalar_prefetch=1,
        grid=(G,),
        in_specs=[
            pl.BlockSpec((TT, D), lambda g, eot: (g, 0)),
            pl.BlockSpec((1, D, F), lambda g, eot: (eot[g], 0, 0)),
            pl.BlockSpec((1, 1, F), lambda g, eot: (eot[g], 0, 0)),
            pl.BlockSpec((1, F, D), lambda g, eot: (eot[g], 0, 0)),
            pl.BlockSpec((1, 1, D), lambda g, eot: (eot[g], 0, 0)),
            pl.BlockSpec((1, 1, D), lambda g, eot: (eot[g], 0, 0)),
            pl.BlockSpec((1, 1, D), lambda g, eot: (eot[g], 0, 0)),
            pl.BlockSpec(memory_space=pltpu.SMEM),
        ],
        out_specs=pl.BlockSpec((TT, D), lambda g, eot: (g, 0)),
    )
    return pl.pallas_call(
        _expert_body,
        grid_spec=grid_spec,
        out_shape=jax.ShapeDtypeStruct((T_PAD, D), jnp.float32),
    )(eot, xg, W1.astype(jnp.bfloat16), b1.reshape(E, 1, F),
      W2.astype(jnp.bfloat16), b2.reshape(E, 1, D),
      ln_g.reshape(E, 1, D), ln_b.reshape(E, 1, D), alpha_v)


def _sc_unperm_body(yg_hbm, dest_hbm, out_hbm,
                    dest_v, dest2_v, rows_v,
                    gsem0, gsem1, gsem2, wsem0, wsem1, wsem2):
    wid = lax.axis_index("c") * NS + lax.axis_index("s")
    lo = wid * TW
    gsems = [gsem0, gsem1, gsem2]
    wsems = [wsem0, wsem1, wsem2]

    pltpu.sync_copy(dest_hbm.at[pl.ds(lo, TW)], dest_v)
    for c in range(CT):
        dest2_v[c, :] = dest_v[pl.ds(c * L, L)]

    gcopies = [None] * CT
    wcopies = [None] * CT
    for c in range(NB):
        gcopies[c] = pltpu.async_copy(yg_hbm.at[dest2_v.at[c]],
                                      rows_v.at[c % NB], gsems[c % NB])
    for c in range(CT):
        b = c % NB
        gcopies[c].wait()
        wcopies[c] = pltpu.async_copy(rows_v.at[b],
                                      out_hbm.at[pl.ds(lo + c * L, L)], wsems[b])
        if c + NB < CT:
            wcopies[c].wait()
            gcopies[c + NB] = pltpu.async_copy(
                yg_hbm.at[dest2_v.at[c + NB]], rows_v.at[b], gsems[b])
    for c in range(max(CT - NB, 0), CT):
        if wcopies[c] is not None:
            wcopies[c].wait()


def _sc_unperm(yg, dest):
    mesh = plsc.VectorSubcoreMesh(core_axis_name="c", subcore_axis_name="s")
    f = functools.partial(
        pl.kernel,
        mesh=mesh,
        compiler_params=pltpu.CompilerParams(needs_layout_passes=False),
        out_type=jax.ShapeDtypeStruct((T, D), jnp.float32),
        scratch_types=[
            pltpu.VMEM((TW,), jnp.int32),
            pltpu.VMEM((CT, L), jnp.int32),
            pltpu.VMEM((NB, L, D), jnp.float32),
            pltpu.SemaphoreType.DMA,
            pltpu.SemaphoreType.DMA,
            pltpu.SemaphoreType.DMA,
            pltpu.SemaphoreType.DMA,
            pltpu.SemaphoreType.DMA,
            pltpu.SemaphoreType.DMA,
        ],
    )(_sc_unperm_body)
    return f(yg, dest)


def kernel(hidden_states, W1, b1, W2, b2, ln_g, ln_b, centroids, alpha):
    x = hidden_states
    dest2, eot2 = _route(x, centroids)
    dest = dest2[:, 0]
    eot = eot2[:, 0]
    xg = _sc_gather(x, dest)
    alpha_v = jnp.reshape(alpha, (1,))
    yg = _expert_mlp(eot, xg, W1, b1, W2, b2, ln_g, ln_b, alpha_v)
    return _sc_unperm(yg, dest)


# speedup vs baseline: 1.0570x; 1.0570x over previous
import functools

import jax
import jax.numpy as jnp
from jax import lax
from jax.experimental import pallas as pl
from jax.experimental.pallas import tpu as pltpu
from jax.experimental.pallas import tpu_sc as plsc

T = 2048
D = 2048
E = 8
F = 128

TT = 128
D2 = D // 2
G = T // TT + E
T_PAD = G * TT

NC = 2
NS = 16
NW = NC * NS
L = 16
SLOTS_W = T_PAD // NW
CH = SLOTS_W // L
DTOK_W = T // NW


def _router_body(x_ref, cn_ref, dest_ref, eot_ref, xb_ref):
    cn = cn_ref[...]
    cn = cn / (jnp.sqrt(jnp.sum(cn * cn, axis=-1, keepdims=True)) + 1e-8)
    cnt = cn.T

    nchunk = T // TT
    ri = lax.broadcasted_iota(jnp.int32, (TT, TT), 0)
    ci = lax.broadcasted_iota(jnp.int32, (TT, TT), 1)
    Lt = jnp.where(ri >= ci, 1.0, 0.0).astype(jnp.float32)

    ohs = []
    incs = []
    tots = []
    for c in range(nchunk):
        xc = x_ref[pl.ds(c * TT, TT), :]
        xci = jax.lax.bitcast_convert_type(xc, jnp.int32)
        rl = xci[:, :D2]
        rh = xci[:, D2:]
        rl = (rl + 0x7FFF + ((rl >> 16) & 1)) >> 16
        rh = (rh + 0x7FFF + ((rh >> 16) & 1)) >> 16
        xb_ref[pl.ds(c * TT, TT), :] = (rh << 16) | (rl & 0xFFFF)
        xn = xc / (jnp.sqrt(jnp.sum(xc * xc, axis=-1, keepdims=True)) + 1e-8)
        sims = jnp.dot(xn, cnt, preferred_element_type=jnp.float32)
        mx = jnp.max(sims, axis=-1, keepdims=True)
        eidx = lax.broadcasted_iota(jnp.int32, (TT, E), 1)
        idx = jnp.min(jnp.where(sims >= mx, eidx, E), axis=-1, keepdims=True)
        oh = jnp.where(eidx == idx, 1.0, 0.0).astype(jnp.float32)
        inc = jnp.dot(Lt, oh, preferred_element_type=jnp.float32)
        ohs.append(oh)
        incs.append(inc)
        tots.append(inc[TT - 1:TT, :])

    tot = jnp.concatenate(tots, axis=0)
    ri2 = lax.broadcasted_iota(jnp.int32, (nchunk, nchunk), 0)
    ci2 = lax.broadcasted_iota(jnp.int32, (nchunk, nchunk), 1)
    Ls = jnp.where(ri2 > ci2, 1.0, 0.0).astype(jnp.float32)
    ex = jnp.dot(Ls, tot, preferred_element_type=jnp.float32)
    counts = ex[nchunk - 1:nchunk, :] + tot[nchunk - 1:nchunk, :]

    counts_i = (counts + 0.5).astype(jnp.int32)
    tiles_i = (counts_i + (TT - 1)) >> 7
    tiles_f = tiles_i.astype(jnp.float32)
    ri3 = lax.broadcasted_iota(jnp.int32, (E, E), 0)
    ci3 = lax.broadcasted_iota(jnp.int32, (E, E), 1)
    Mu = jnp.where(ri3 <= ci3, 1.0, 0.0).astype(jnp.float32)
    cum_tiles = jnp.dot(tiles_f, Mu, preferred_element_type=jnp.float32)
    poff = (cum_tiles - tiles_f) * float(TT)

    for c in range(nchunk):
        base = poff + ex[c:c + 1, :]
        dvals = jnp.sum(ohs[c] * (base + incs[c] - 1.0), axis=-1, keepdims=True)
        di = dvals.astype(jnp.int32)
        di = (di & ~127) | ((di & 127) ^ 83)
        dest_ref[pl.ds(c * TT, TT), :] = di

    gio = lax.broadcasted_iota(jnp.int32, (G, E), 0).astype(jnp.float32)
    ge = jnp.where(gio >= cum_tiles, 1.0, 0.0)
    eot = jnp.sum(ge, axis=-1, keepdims=True).astype(jnp.int32)
    eot_ref[...] = jnp.minimum(eot, E - 1)


def _route(x, centroids):
    return pl.pallas_call(
        _router_body,
        grid=(1,),
        in_specs=[
            pl.BlockSpec((T, D), lambda i: (0, 0)),
            pl.BlockSpec((E, D), lambda i: (0, 0)),
        ],
        out_specs=[
            pl.BlockSpec((T, 1), lambda i: (0, 0)),
            pl.BlockSpec((G, 1), lambda i: (0, 0)),
            pl.BlockSpec((T, D2), lambda i: (0, 0)),
        ],
        out_shape=[
            jax.ShapeDtypeStruct((T, 1), jnp.int32),
            jax.ShapeDtypeStruct((G, 1), jnp.int32),
            jax.ShapeDtypeStruct((T, D2), jnp.int32),
        ],
    )(x, centroids)


NB = 3
TW = T // NW
CT = TW // L


def _sc_gather_body(x_hbm, dest_hbm, xg_hbm,
                    dest_v, dest2_v, rows_v,
                    gsem0, gsem1, gsem2, wsem0, wsem1, wsem2):
    wid = lax.axis_index("c") * NS + lax.axis_index("s")
    lo = wid * TW
    gsems = [gsem0, gsem1, gsem2]
    wsems = [wsem0, wsem1, wsem2]

    pltpu.sync_copy(dest_hbm.at[pl.ds(lo, TW)], dest_v)
    for c in range(CT):
        dest2_v[c, :] = dest_v[pl.ds(c * L, L)]

    gcopies = [None] * CT
    wcopies = [None] * CT
    for c in range(NB):
        gcopies[c] = pltpu.async_copy(x_hbm.at[pl.ds(lo + c * L, L)],
                                      rows_v.at[c % NB], gsems[c % NB])
    for c in range(CT):
        b = c % NB
        gcopies[c].wait()
        wcopies[c] = pltpu.async_copy(rows_v.at[b],
                                      xg_hbm.at[dest2_v.at[c]], wsems[b])
        if c + NB < CT:
            wcopies[c].wait()
            gcopies[c + NB] = pltpu.async_copy(
                x_hbm.at[pl.ds(lo + (c + NB) * L, L)], rows_v.at[b], gsems[b])
    for c in range(max(CT - NB, 0), CT):
        if wcopies[c] is not None:
            wcopies[c].wait()


def _sc_gather(x, dest):
    mesh = plsc.VectorSubcoreMesh(core_axis_name="c", subcore_axis_name="s")
    f = functools.partial(
        pl.kernel,
        mesh=mesh,
        compiler_params=pltpu.CompilerParams(needs_layout_passes=False),
        out_type=jax.ShapeDtypeStruct((T_PAD, D2), jnp.int32),
        scratch_types=[
            pltpu.VMEM((TW,), jnp.int32),
            pltpu.VMEM((CT, L), jnp.int32),
            pltpu.VMEM((NB, L, D2), jnp.int32),
            pltpu.SemaphoreType.DMA,
            pltpu.SemaphoreType.DMA,
            pltpu.SemaphoreType.DMA,
            pltpu.SemaphoreType.DMA,
            pltpu.SemaphoreType.DMA,
            pltpu.SemaphoreType.DMA,
        ],
    )(_sc_gather_body)
    return f(x, dest)


def _expert_body(eot_ref, xg_ref, W1_ref, b1_ref, W2_ref, b2_ref,
                 lng_ref, lnb_ref, alpha_ref, out_ref):
    xi = xg_ref[...]
    f_lo = jax.lax.bitcast_convert_type(xi << 16, jnp.float32)
    f_hi = jax.lax.bitcast_convert_type((xi >> 16) << 16, jnp.float32)
    x = jnp.concatenate([f_lo, f_hi], axis=1)
    h = jnp.dot(x, W1_ref[0], preferred_element_type=jnp.float32) + b1_ref[0]
    h = 0.5 * h * (1.0 + lax.erf(h * 0.7071067811865476))
    y = jnp.dot(h, W2_ref[0], preferred_element_type=jnp.float32) + b2_ref[0]
    mu = jnp.mean(y, axis=-1, keepdims=True)
    yc = y - mu
    var = jnp.mean(yc * yc, axis=-1, keepdims=True)
    y_ln = yc * lax.rsqrt(var + 1e-5) * lng_ref[0] + lnb_ref[0]
    out_ref[...] = x + alpha_ref[0] * y_ln


def _expert_mlp(eot, xg, W1, b1, W2, b2, ln_g, ln_b, alpha_v):
    grid_spec = pltpu.PrefetchScalarGridSpec(
        num_scalar_prefetch=1,
        grid=(G,),
        in_specs=[
            pl.BlockSpec((TT, D2), lambda g, eot: (g, 0)),
            pl.BlockSpec((1, D, F), lambda g, eot: (eot[g], 0, 0)),
            pl.BlockSpec((1, 1, F), lambda g, eot: (eot[g], 0, 0)),
            pl.BlockSpec((1, F, D), lambda g, eot: (eot[g], 0, 0)),
            pl.BlockSpec((1, 1, D), lambda g, eot: (eot[g], 0, 0)),
            pl.BlockSpec((1, 1, D), lambda g, eot: (eot[g], 0, 0)),
            pl.BlockSpec((1, 1, D), lambda g, eot: (eot[g], 0, 0)),
            pl.BlockSpec(memory_space=pltpu.SMEM),
        ],
        out_specs=pl.BlockSpec((TT, D), lambda g, eot: (g, 0)),
    )
    return pl.pallas_call(
        _expert_body,
        grid_spec=grid_spec,
        out_shape=jax.ShapeDtypeStruct((T_PAD, D), jnp.float32),
    )(eot, xg, W1, b1.reshape(E, 1, F), W2, b2.reshape(E, 1, D),
      ln_g.reshape(E, 1, D), ln_b.reshape(E, 1, D), alpha_v)


def _sc_unperm_body(yg_hbm, dest_hbm, out_hbm,
                    dest_v, dest2_v, rows_v,
                    gsem0, gsem1, gsem2, wsem0, wsem1, wsem2):
    wid = lax.axis_index("c") * NS + lax.axis_index("s")
    lo = wid * TW
    gsems = [gsem0, gsem1, gsem2]
    wsems = [wsem0, wsem1, wsem2]

    pltpu.sync_copy(dest_hbm.at[pl.ds(lo, TW)], dest_v)
    for c in range(CT):
        dest2_v[c, :] = dest_v[pl.ds(c * L, L)]

    gcopies = [None] * CT
    wcopies = [None] * CT
    for c in range(NB):
        gcopies[c] = pltpu.async_copy(yg_hbm.at[dest2_v.at[c]],
                                      rows_v.at[c % NB], gsems[c % NB])
    for c in range(CT):
        b = c % NB
        gcopies[c].wait()
        wcopies[c] = pltpu.async_copy(rows_v.at[b],
                                      out_hbm.at[pl.ds(lo + c * L, L)], wsems[b])
        if c + NB < CT:
            wcopies[c].wait()
            gcopies[c + NB] = pltpu.async_copy(
                yg_hbm.at[dest2_v.at[c + NB]], rows_v.at[b], gsems[b])
    for c in range(max(CT - NB, 0), CT):
        if wcopies[c] is not None:
            wcopies[c].wait()


def _sc_unperm(yg, dest):
    mesh = plsc.VectorSubcoreMesh(core_axis_name="c", subcore_axis_name="s")
    f = functools.partial(
        pl.kernel,
        mesh=mesh,
        compiler_params=pltpu.CompilerParams(needs_layout_passes=False),
        out_type=jax.ShapeDtypeStruct((T, D), jnp.float32),
        scratch_types=[
            pltpu.VMEM((TW,), jnp.int32),
            pltpu.VMEM((CT, L), jnp.int32),
            pltpu.VMEM((NB, L, D), jnp.float32),
            pltpu.SemaphoreType.DMA,
            pltpu.SemaphoreType.DMA,
            pltpu.SemaphoreType.DMA,
            pltpu.SemaphoreType.DMA,
            pltpu.SemaphoreType.DMA,
            pltpu.SemaphoreType.DMA,
        ],
    )(_sc_unperm_body)
    return f(yg, dest)


def kernel(hidden_states, W1, b1, W2, b2, ln_g, ln_b, centroids, alpha):
    x = hidden_states
    dest2, eot2, xb = _route(x, centroids)
    dest = dest2[:, 0]
    eot = eot2[:, 0]
    xg = _sc_gather(xb, dest)
    alpha_v = jnp.reshape(alpha, (1,))
    yg = _expert_mlp(eot, xg, W1, b1, W2, b2, ln_g, ln_b, alpha_v)
    return _sc_unperm(yg, dest)


# speedup vs baseline: 1.0716x; 1.0138x over previous
import functools

import jax
import jax.numpy as jnp
from jax import lax
from jax.experimental import pallas as pl
from jax.experimental.pallas import tpu as pltpu
from jax.experimental.pallas import tpu_sc as plsc

T = 2048
D = 2048
E = 8
F = 128

TT = 128
D2 = D // 2
G = T // TT + E
T_PAD = G * TT

NC = 2
NS = 16
NW = NC * NS
L = 16
SLOTS_W = T_PAD // NW
CH = SLOTS_W // L
DTOK_W = T // NW


def _router_body(x_ref, cn_ref, dest_ref, eot_ref, xb_ref):
    cn = cn_ref[...]
    cn = cn / (jnp.sqrt(jnp.sum(cn * cn, axis=-1, keepdims=True)) + 1e-8)
    cnt = cn.T

    nchunk = T // TT
    ri = lax.broadcasted_iota(jnp.int32, (TT, TT), 0)
    ci = lax.broadcasted_iota(jnp.int32, (TT, TT), 1)
    Lt = jnp.where(ri >= ci, 1.0, 0.0).astype(jnp.float32)

    ohs = []
    incs = []
    tots = []
    for c in range(nchunk):
        xc = x_ref[pl.ds(c * TT, TT), :]
        xci = jax.lax.bitcast_convert_type(xc, jnp.int32)
        rl = xci[:, :D2]
        rh = xci[:, D2:]
        rl = (rl + 0x7FFF + ((rl >> 16) & 1)) >> 16
        rh = (rh + 0x7FFF + ((rh >> 16) & 1)) >> 16
        xb_ref[pl.ds(c * TT, TT), :] = (rh << 16) | (rl & 0xFFFF)
        xn = xc / (jnp.sqrt(jnp.sum(xc * xc, axis=-1, keepdims=True)) + 1e-8)
        sims = jnp.dot(xn, cnt, preferred_element_type=jnp.float32)
        mx = jnp.max(sims, axis=-1, keepdims=True)
        eidx = lax.broadcasted_iota(jnp.int32, (TT, E), 1)
        idx = jnp.min(jnp.where(sims >= mx, eidx, E), axis=-1, keepdims=True)
        oh = jnp.where(eidx == idx, 1.0, 0.0).astype(jnp.float32)
        inc = jnp.dot(Lt, oh, preferred_element_type=jnp.float32)
        ohs.append(oh)
        incs.append(inc)
        tots.append(inc[TT - 1:TT, :])

    tot = jnp.concatenate(tots, axis=0)
    ri2 = lax.broadcasted_iota(jnp.int32, (nchunk, nchunk), 0)
    ci2 = lax.broadcasted_iota(jnp.int32, (nchunk, nchunk), 1)
    Ls = jnp.where(ri2 > ci2, 1.0, 0.0).astype(jnp.float32)
    ex = jnp.dot(Ls, tot, preferred_element_type=jnp.float32)
    counts = ex[nchunk - 1:nchunk, :] + tot[nchunk - 1:nchunk, :]

    counts_i = (counts + 0.5).astype(jnp.int32)
    tiles_i = (counts_i + (TT - 1)) >> 7
    tiles_f = tiles_i.astype(jnp.float32)
    ri3 = lax.broadcasted_iota(jnp.int32, (E, E), 0)
    ci3 = lax.broadcasted_iota(jnp.int32, (E, E), 1)
    Mu = jnp.where(ri3 <= ci3, 1.0, 0.0).astype(jnp.float32)
    cum_tiles = jnp.dot(tiles_f, Mu, preferred_element_type=jnp.float32)
    poff = (cum_tiles - tiles_f) * float(TT)

    for c in range(nchunk):
        base = poff + ex[c:c + 1, :]
        dvals = jnp.sum(ohs[c] * (base + incs[c] - 1.0), axis=-1, keepdims=True)
        di = dvals.astype(jnp.int32)
        di = (di & ~127) | ((di & 127) ^ 83)
        dest_ref[pl.ds(c * TT, TT), :] = di

    gio = lax.broadcasted_iota(jnp.int32, (G, E), 0).astype(jnp.float32)
    ge = jnp.where(gio >= cum_tiles, 1.0, 0.0)
    eot = jnp.sum(ge, axis=-1, keepdims=True).astype(jnp.int32)
    eot_ref[...] = jnp.minimum(eot, E - 1)


def _route(x, centroids):
    return pl.pallas_call(
        _router_body,
        grid=(1,),
        in_specs=[
            pl.BlockSpec((T, D), lambda i: (0, 0)),
            pl.BlockSpec((E, D), lambda i: (0, 0)),
        ],
        out_specs=[
            pl.BlockSpec((T, 1), lambda i: (0, 0)),
            pl.BlockSpec((G, 1), lambda i: (0, 0)),
            pl.BlockSpec((T, D2), lambda i: (0, 0)),
        ],
        out_shape=[
            jax.ShapeDtypeStruct((T, 1), jnp.int32),
            jax.ShapeDtypeStruct((G, 1), jnp.int32),
            jax.ShapeDtypeStruct((T, D2), jnp.int32),
        ],
    )(x, centroids)


NB = 3
TW = T // NW
CT = TW // L


def _sc_gather_body(x_hbm, dest_hbm, xg_hbm,
                    dest_v, dest2_v, rows_v,
                    gsem0, gsem1, gsem2, wsem0, wsem1, wsem2):
    wid = lax.axis_index("c") * NS + lax.axis_index("s")
    lo = wid * TW
    gsems = [gsem0, gsem1, gsem2]
    wsems = [wsem0, wsem1, wsem2]

    pltpu.sync_copy(dest_hbm.at[pl.ds(lo, TW)], dest_v)
    for c in range(CT):
        dest2_v[c, :] = dest_v[pl.ds(c * L, L)]

    gcopies = [None] * CT
    wcopies = [None] * CT
    for c in range(NB):
        gcopies[c] = pltpu.async_copy(x_hbm.at[pl.ds(lo + c * L, L)],
                                      rows_v.at[c % NB], gsems[c % NB])
    for c in range(CT):
        b = c % NB
        gcopies[c].wait()
        wcopies[c] = pltpu.async_copy(rows_v.at[b],
                                      xg_hbm.at[dest2_v.at[c]], wsems[b])
        if c + NB < CT:
            wcopies[c].wait()
            gcopies[c + NB] = pltpu.async_copy(
                x_hbm.at[pl.ds(lo + (c + NB) * L, L)], rows_v.at[b], gsems[b])
    for c in range(max(CT - NB, 0), CT):
        if wcopies[c] is not None:
            wcopies[c].wait()


def _sc_gather(x, dest):
    mesh = plsc.VectorSubcoreMesh(core_axis_name="c", subcore_axis_name="s")
    f = functools.partial(
        pl.kernel,
        mesh=mesh,
        compiler_params=pltpu.CompilerParams(needs_layout_passes=False),
        out_type=jax.ShapeDtypeStruct((T_PAD, D2), jnp.int32),
        scratch_types=[
            pltpu.VMEM((TW,), jnp.int32),
            pltpu.VMEM((CT, L), jnp.int32),
            pltpu.VMEM((NB, L, D2), jnp.int32),
            pltpu.SemaphoreType.DMA,
            pltpu.SemaphoreType.DMA,
            pltpu.SemaphoreType.DMA,
            pltpu.SemaphoreType.DMA,
            pltpu.SemaphoreType.DMA,
            pltpu.SemaphoreType.DMA,
        ],
    )(_sc_gather_body)
    return f(x, dest)


def _expert_body(eot_ref, xg_ref, W1_ref, b1_ref, W2_ref, b2_ref,
                 lng_ref, lnb_ref, alpha_ref, out_ref):
    e = eot_ref[pl.program_id(0)]
    xi = xg_ref[...]
    f_lo = jax.lax.bitcast_convert_type(xi << 16, jnp.float32)
    f_hi = jax.lax.bitcast_convert_type((xi >> 16) << 16, jnp.float32)
    x = jnp.concatenate([f_lo, f_hi], axis=1)
    h = jnp.dot(x, W1_ref[0], preferred_element_type=jnp.float32) + b1_ref[pl.ds(e, 1)]
    h = 0.5 * h * (1.0 + lax.erf(h * 0.7071067811865476))
    y = jnp.dot(h, W2_ref[0], preferred_element_type=jnp.float32) + b2_ref[pl.ds(e, 1)]
    mu = jnp.mean(y, axis=-1, keepdims=True)
    yc = y - mu
    var = jnp.mean(yc * yc, axis=-1, keepdims=True)
    y_ln = yc * lax.rsqrt(var + 1e-5) * lng_ref[pl.ds(e, 1)] + lnb_ref[pl.ds(e, 1)]
    out_ref[...] = x + alpha_ref[0] * y_ln


def _expert_mlp(eot, xg, W1, b1, W2, b2, ln_g, ln_b, alpha_v):
    grid_spec = pltpu.PrefetchScalarGridSpec(
        num_scalar_prefetch=1,
        grid=(G,),
        in_specs=[
            pl.BlockSpec((TT, D2), lambda g, eot: (g, 0)),
            pl.BlockSpec((1, D, F), lambda g, eot: (eot[g], 0, 0)),
            pl.BlockSpec((E, F), lambda g, eot: (0, 0)),
            pl.BlockSpec((1, F, D), lambda g, eot: (eot[g], 0, 0)),
            pl.BlockSpec((E, D), lambda g, eot: (0, 0)),
            pl.BlockSpec((E, D), lambda g, eot: (0, 0)),
            pl.BlockSpec((E, D), lambda g, eot: (0, 0)),
            pl.BlockSpec(memory_space=pltpu.SMEM),
        ],
        out_specs=pl.BlockSpec((TT, D), lambda g, eot: (g, 0)),
    )
    return pl.pallas_call(
        _expert_body,
        grid_spec=grid_spec,
        out_shape=jax.ShapeDtypeStruct((T_PAD, D), jnp.float32),
    )(eot, xg, W1, b1, W2, b2, ln_g, ln_b, alpha_v)


def _sc_unperm_body(yg_hbm, dest_hbm, out_hbm,
                    dest_v, dest2_v, rows_v,
                    gsem0, gsem1, gsem2, wsem0, wsem1, wsem2):
    wid = lax.axis_index("c") * NS + lax.axis_index("s")
    lo = wid * TW
    gsems = [gsem0, gsem1, gsem2]
    wsems = [wsem0, wsem1, wsem2]

    pltpu.sync_copy(dest_hbm.at[pl.ds(lo, TW)], dest_v)
    for c in range(CT):
        dest2_v[c, :] = dest_v[pl.ds(c * L, L)]

    gcopies = [None] * CT
    wcopies = [None] * CT
    for c in range(NB):
        gcopies[c] = pltpu.async_copy(yg_hbm.at[dest2_v.at[c]],
                                      rows_v.at[c % NB], gsems[c % NB])
    for c in range(CT):
        b = c % NB
        gcopies[c].wait()
        wcopies[c] = pltpu.async_copy(rows_v.at[b],
                                      out_hbm.at[pl.ds(lo + c * L, L)], wsems[b])
        if c + NB < CT:
            wcopies[c].wait()
            gcopies[c + NB] = pltpu.async_copy(
                yg_hbm.at[dest2_v.at[c + NB]], rows_v.at[b], gsems[b])
    for c in range(max(CT - NB, 0), CT):
        if wcopies[c] is not None:
            wcopies[c].wait()


def _sc_unperm(yg, dest):
    mesh = plsc.VectorSubcoreMesh(core_axis_name="c", subcore_axis_name="s")
    f = functools.partial(
        pl.kernel,
        mesh=mesh,
        compiler_params=pltpu.CompilerParams(needs_layout_passes=False),
        out_type=jax.ShapeDtypeStruct((T, D), jnp.float32),
        scratch_types=[
            pltpu.VMEM((TW,), jnp.int32),
            pltpu.VMEM((CT, L), jnp.int32),
            pltpu.VMEM((NB, L, D), jnp.float32),
            pltpu.SemaphoreType.DMA,
            pltpu.SemaphoreType.DMA,
            pltpu.SemaphoreType.DMA,
            pltpu.SemaphoreType.DMA,
            pltpu.SemaphoreType.DMA,
            pltpu.SemaphoreType.DMA,
        ],
    )(_sc_unperm_body)
    return f(yg, dest)


def kernel(hidden_states, W1, b1, W2, b2, ln_g, ln_b, centroids, alpha):
    x = hidden_states
    dest2, eot2, xb = _route(x, centroids)
    dest = jnp.reshape(dest2, (T,))
    eot = jnp.reshape(eot2, (G,))
    xg = _sc_gather(xb, dest)
    alpha_v = jnp.reshape(alpha, (1,))
    yg = _expert_mlp(eot, xg, W1, b1, W2, b2, ln_g, ln_b, alpha_v)
    return _sc_unperm(yg, dest)


# speedup vs baseline: 1.0785x; 1.0065x over previous
import functools

import jax
import jax.numpy as jnp
from jax import lax
from jax.experimental import pallas as pl
from jax.experimental.pallas import tpu as pltpu
from jax.experimental.pallas import tpu_sc as plsc

T = 2048
D = 2048
E = 8
F = 128

TT = 128
D2 = D // 2
G = T // TT + E
T_PAD = G * TT

NC = 2
NS = 16
NW = NC * NS
L = 16
SLOTS_W = T_PAD // NW
CH = SLOTS_W // L
DTOK_W = T // NW


def _router_body(x_ref, cn_ref, dest_ref, eot_ref, xb_ref):
    cn = cn_ref[...]
    cn = cn / (jnp.sqrt(jnp.sum(cn * cn, axis=-1, keepdims=True)) + 1e-8)
    cnt = cn.T

    nchunk = T // TT
    ri = lax.broadcasted_iota(jnp.int32, (TT, TT), 0)
    ci = lax.broadcasted_iota(jnp.int32, (TT, TT), 1)
    Lt = jnp.where(ri >= ci, 1.0, 0.0).astype(jnp.float32)

    ohs = []
    incs = []
    tots = []
    for c in range(nchunk):
        xc = x_ref[pl.ds(c * TT, TT), :]
        xci = jax.lax.bitcast_convert_type(xc, jnp.int32)
        rl = xci[:, :D2]
        rh = xci[:, D2:]
        rl = (rl + 0x7FFF + ((rl >> 16) & 1)) >> 16
        rh = (rh + 0x7FFF + ((rh >> 16) & 1)) >> 16
        xb_ref[pl.ds(c * TT, TT), :] = (rh << 16) | (rl & 0xFFFF)
        xn = xc / (jnp.sqrt(jnp.sum(xc * xc, axis=-1, keepdims=True)) + 1e-8)
        sims = jnp.dot(xn, cnt, preferred_element_type=jnp.float32)
        mx = jnp.max(sims, axis=-1, keepdims=True)
        eidx = lax.broadcasted_iota(jnp.int32, (TT, E), 1)
        idx = jnp.min(jnp.where(sims >= mx, eidx, E), axis=-1, keepdims=True)
        oh = jnp.where(eidx == idx, 1.0, 0.0).astype(jnp.float32)
        inc = jnp.dot(Lt, oh, preferred_element_type=jnp.float32)
        ohs.append(oh)
        incs.append(inc)
        tots.append(inc[TT - 1:TT, :])

    tot = jnp.concatenate(tots, axis=0)
    ri2 = lax.broadcasted_iota(jnp.int32, (nchunk, nchunk), 0)
    ci2 = lax.broadcasted_iota(jnp.int32, (nchunk, nchunk), 1)
    Ls = jnp.where(ri2 > ci2, 1.0, 0.0).astype(jnp.float32)
    ex = jnp.dot(Ls, tot, preferred_element_type=jnp.float32)
    counts = ex[nchunk - 1:nchunk, :] + tot[nchunk - 1:nchunk, :]

    counts_i = (counts + 0.5).astype(jnp.int32)
    tiles_i = (counts_i + (TT - 1)) >> 7
    tiles_f = tiles_i.astype(jnp.float32)
    ri3 = lax.broadcasted_iota(jnp.int32, (E, E), 0)
    ci3 = lax.broadcasted_iota(jnp.int32, (E, E), 1)
    Mu = jnp.where(ri3 <= ci3, 1.0, 0.0).astype(jnp.float32)
    cum_tiles = jnp.dot(tiles_f, Mu, preferred_element_type=jnp.float32)
    poff = (cum_tiles - tiles_f) * float(TT)

    for c in range(nchunk):
        base = poff + ex[c:c + 1, :]
        dvals = jnp.sum(ohs[c] * (base + incs[c] - 1.0), axis=-1, keepdims=True)
        di = dvals.astype(jnp.int32)
        di = (di & ~127) | ((di & 127) ^ 83)
        dest_ref[pl.ds(c * TT, TT), :] = di

    gio = lax.broadcasted_iota(jnp.int32, (G, E), 0).astype(jnp.float32)
    ge = jnp.where(gio >= cum_tiles, 1.0, 0.0)
    eot = jnp.sum(ge, axis=-1, keepdims=True).astype(jnp.int32)
    eot_ref[...] = jnp.minimum(eot, E - 1)


def _route(x, centroids):
    return pl.pallas_call(
        _router_body,
        grid=(1,),
        in_specs=[
            pl.BlockSpec((T, D), lambda i: (0, 0)),
            pl.BlockSpec((E, D), lambda i: (0, 0)),
        ],
        out_specs=[
            pl.BlockSpec((T, 1), lambda i: (0, 0)),
            pl.BlockSpec((G, 1), lambda i: (0, 0)),
            pl.BlockSpec((T, D2), lambda i: (0, 0)),
        ],
        out_shape=[
            jax.ShapeDtypeStruct((T, 1), jnp.int32),
            jax.ShapeDtypeStruct((G, 1), jnp.int32),
            jax.ShapeDtypeStruct((T, D2), jnp.int32),
        ],
    )(x, centroids)


NB = 3
TW = T // NW
CT = TW // L


def _sc_gather_body(x_hbm, dest_hbm, xg_hbm,
                    dest_v, dest2_v, rows_v,
                    gsem0, gsem1, gsem2, wsem0, wsem1, wsem2):
    wid = lax.axis_index("c") * NS + lax.axis_index("s")
    lo = wid * TW
    gsems = [gsem0, gsem1, gsem2]
    wsems = [wsem0, wsem1, wsem2]

    pltpu.sync_copy(dest_hbm.at[pl.ds(lo, TW)], dest_v)
    for c in range(CT):
        dest2_v[c, :] = dest_v[pl.ds(c * L, L)]

    gcopies = [None] * CT
    wcopies = [None] * CT
    for c in range(NB):
        gcopies[c] = pltpu.async_copy(x_hbm.at[pl.ds(lo + c * L, L)],
                                      rows_v.at[c % NB], gsems[c % NB])
    for c in range(CT):
        b = c % NB
        gcopies[c].wait()
        wcopies[c] = pltpu.async_copy(rows_v.at[b],
                                      xg_hbm.at[dest2_v.at[c]], wsems[b])
        if c + NB < CT:
            wcopies[c].wait()
            gcopies[c + NB] = pltpu.async_copy(
                x_hbm.at[pl.ds(lo + (c + NB) * L, L)], rows_v.at[b], gsems[b])
    for c in range(max(CT - NB, 0), CT):
        if wcopies[c] is not None:
            wcopies[c].wait()


def _sc_gather(x, dest):
    mesh = plsc.VectorSubcoreMesh(core_axis_name="c", subcore_axis_name="s")
    f = functools.partial(
        pl.kernel,
        mesh=mesh,
        compiler_params=pltpu.CompilerParams(needs_layout_passes=False),
        out_type=jax.ShapeDtypeStruct((T_PAD, D2), jnp.int32),
        scratch_types=[
            pltpu.VMEM((TW,), jnp.int32),
            pltpu.VMEM((CT, L), jnp.int32),
            pltpu.VMEM((NB, L, D2), jnp.int32),
            pltpu.SemaphoreType.DMA,
            pltpu.SemaphoreType.DMA,
            pltpu.SemaphoreType.DMA,
            pltpu.SemaphoreType.DMA,
            pltpu.SemaphoreType.DMA,
            pltpu.SemaphoreType.DMA,
        ],
    )(_sc_gather_body)
    return f(x, dest)


def _expert_body(eot_ref, xg_ref, W1_ref, b1_ref, W2_ref, b2_ref,
                 lng_ref, lnb_ref, alpha_ref, out_ref):
    e = eot_ref[pl.program_id(0)]
    xi = xg_ref[...]
    f_lo = jax.lax.bitcast_convert_type(xi << 16, jnp.float32)
    f_hi = jax.lax.bitcast_convert_type((xi >> 16) << 16, jnp.float32)
    x = jnp.concatenate([f_lo, f_hi], axis=1)
    h = jnp.dot(x, W1_ref[e], preferred_element_type=jnp.float32) + b1_ref[pl.ds(e, 1)]
    h = 0.5 * h * (1.0 + lax.erf(h * 0.7071067811865476))
    y = jnp.dot(h, W2_ref[e], preferred_element_type=jnp.float32) + b2_ref[pl.ds(e, 1)]
    mu = jnp.mean(y, axis=-1, keepdims=True)
    yc = y - mu
    var = jnp.mean(yc * yc, axis=-1, keepdims=True)
    y_ln = yc * lax.rsqrt(var + 1e-5) * lng_ref[pl.ds(e, 1)] + lnb_ref[pl.ds(e, 1)]
    out_ref[...] = x + alpha_ref[0] * y_ln


def _expert_mlp(eot, xg, W1, b1, W2, b2, ln_g, ln_b, alpha_v):
    grid_spec = pltpu.PrefetchScalarGridSpec(
        num_scalar_prefetch=1,
        grid=(G,),
        in_specs=[
            pl.BlockSpec((TT, D2), lambda g, eot: (g, 0)),
            pl.BlockSpec((E, D, F), lambda g, eot: (0, 0, 0)),
            pl.BlockSpec((E, F), lambda g, eot: (0, 0)),
            pl.BlockSpec((E, F, D), lambda g, eot: (0, 0, 0)),
            pl.BlockSpec((E, D), lambda g, eot: (0, 0)),
            pl.BlockSpec((E, D), lambda g, eot: (0, 0)),
            pl.BlockSpec((E, D), lambda g, eot: (0, 0)),
            pl.BlockSpec(memory_space=pltpu.SMEM),
        ],
        out_specs=pl.BlockSpec((TT, D), lambda g, eot: (g, 0)),
    )
    return pl.pallas_call(
        _expert_body,
        grid_spec=grid_spec,
        out_shape=jax.ShapeDtypeStruct((T_PAD, D), jnp.float32),
    )(eot, xg, W1, b1, W2, b2, ln_g, ln_b, alpha_v)


def _sc_unperm_body(yg_hbm, dest_hbm, out_hbm,
                    dest_v, dest2_v, rows_v,
                    gsem0, gsem1, gsem2, wsem0, wsem1, wsem2):
    wid = lax.axis_index("c") * NS + lax.axis_index("s")
    lo = wid * TW
    gsems = [gsem0, gsem1, gsem2]
    wsems = [wsem0, wsem1, wsem2]

    pltpu.sync_copy(dest_hbm.at[pl.ds(lo, TW)], dest_v)
    for c in range(CT):
        dest2_v[c, :] = dest_v[pl.ds(c * L, L)]

    gcopies = [None] * CT
    wcopies = [None] * CT
    for c in range(NB):
        gcopies[c] = pltpu.async_copy(yg_hbm.at[dest2_v.at[c]],
                                      rows_v.at[c % NB], gsems[c % NB])
    for c in range(CT):
        b = c % NB
        gcopies[c].wait()
        wcopies[c] = pltpu.async_copy(rows_v.at[b],
                                      out_hbm.at[pl.ds(lo + c * L, L)], wsems[b])
        if c + NB < CT:
            wcopies[c].wait()
            gcopies[c + NB] = pltpu.async_copy(
                yg_hbm.at[dest2_v.at[c + NB]], rows_v.at[b], gsems[b])
    for c in range(max(CT - NB, 0), CT):
        if wcopies[c] is not None:
            wcopies[c].wait()


def _sc_unperm(yg, dest):
    mesh = plsc.VectorSubcoreMesh(core_axis_name="c", subcore_axis_name="s")
    f = functools.partial(
        pl.kernel,
        mesh=mesh,
        compiler_params=pltpu.CompilerParams(needs_layout_passes=False),
        out_type=jax.ShapeDtypeStruct((T, D), jnp.float32),
        scratch_types=[
            pltpu.VMEM((TW,), jnp.int32),
            pltpu.VMEM((CT, L), jnp.int32),
            pltpu.VMEM((NB, L, D), jnp.float32),
            pltpu.SemaphoreType.DMA,
            pltpu.SemaphoreType.DMA,
            pltpu.SemaphoreType.DMA,
            pltpu.SemaphoreType.DMA,
            pltpu.SemaphoreType.DMA,
            pltpu.SemaphoreType.DMA,
        ],
    )(_sc_unperm_body)
    return f(yg, dest)


def kernel(hidden_states, W1, b1, W2, b2, ln_g, ln_b, centroids, alpha):
    x = hidden_states
    dest2, eot2, xb = _route(x, centroids)
    dest = jnp.reshape(dest2, (T,))
    eot = jnp.reshape(eot2, (G,))
    xg = _sc_gather(xb, dest)
    alpha_v = jnp.reshape(alpha, (1,))
    yg = _expert_mlp(eot, xg, W1, b1, W2, b2, ln_g, ln_b, alpha_v)
    return _sc_unperm(yg, dest)
